# baseline (speedup 1.0000x reference)
import functools

import jax
import jax.numpy as jnp
from jax import lax
from jax.experimental import pallas as pl
from jax.experimental.pallas import tpu as pltpu
from jax.experimental.pallas import tpu_sc as plsc

N = 10000
E = 320000
HID = 128
NUM_NODE_TYPES = 8
NUM_CLASSES = 10

NC = 2
NS = 16
NW = NC * NS
CHUNK = 128
CHUNKS = 79
E_PAD = NW * CHUNKS * CHUNK
NPAD = 10240
DUMMY = NPAD - 8
RPT = NPAD // NS

_mesh = plsc.VectorSubcoreMesh(
    core_axis_name="c", subcore_axis_name="s", num_cores=NC, num_subcores=NS
)


@functools.partial(
    pl.kernel,
    out_type=jax.ShapeDtypeStruct((NC * NPAD,), jnp.float32),
    mesh=_mesh,
    scratch_types=[
        pltpu.VMEM((CHUNKS, CHUNK), jnp.int32),
        pltpu.VMEM((CHUNK,), jnp.float32),
        pltpu.VMEM_SHARED((NPAD,), jnp.float32),
    ],
)
def _deg_kernel(dst_hbm, ones_hbm, zeros_hbm, out_hbm, dst_v, ones_v, deg_sh):
    cid = lax.axis_index("c")
    sid = lax.axis_index("s")
    w = cid * NS + sid
    r0 = sid * RPT
    pltpu.sync_copy(zeros_hbm.at[pl.ds(r0, RPT)], deg_sh.at[pl.ds(r0, RPT)])
    pltpu.sync_copy(ones_hbm, ones_v)
    pltpu.sync_copy(dst_hbm.at[w], dst_v)
    plsc.subcore_barrier()

    def body(j, c):
        pltpu.sync_copy(ones_v, deg_sh.at[dst_v.at[j]], add=True)
        return c

    lax.fori_loop(0, CHUNKS, body, 0)
    plsc.subcore_barrier()
    pltpu.sync_copy(deg_sh.at[pl.ds(r0, RPT)],
                    out_hbm.at[pl.ds(cid * NPAD + r0, RPT)])


@functools.partial(
    pl.kernel,
    out_type=jax.ShapeDtypeStruct((NC * NPAD, HID), jnp.float32),
    mesh=_mesh,
    scratch_types=[
        pltpu.VMEM((CHUNKS, CHUNK), jnp.int32),
        pltpu.VMEM((CHUNKS, CHUNK), jnp.int32),
        pltpu.VMEM((CHUNK, HID), jnp.float32),
        pltpu.VMEM_SHARED((NPAD, HID), jnp.float32),
        pltpu.SemaphoreType.DMA,
    ],
)
def _agg_kernel(g_hbm, src_hbm, dst_hbm, zeros_hbm, out_hbm,
                src_v, dst_v, rows_v, acc_sh, sem):
    cid = lax.axis_index("c")
    sid = lax.axis_index("s")
    w = cid * NS + sid
    r0 = sid * RPT
    pltpu.sync_copy(zeros_hbm.at[pl.ds(r0, RPT)], acc_sh.at[pl.ds(r0, RPT)])
    pltpu.sync_copy(src_hbm.at[w], src_v)
    pltpu.sync_copy(dst_hbm.at[w], dst_v)
    plsc.subcore_barrier()

    def body(j, c):
        pltpu.async_copy(g_hbm.at[src_v.at[j]], rows_v, sem).wait()
        pltpu.sync_copy(rows_v, acc_sh.at[dst_v.at[j]], add=True)
        return c

    lax.fori_loop(0, CHUNKS, body, 0)
    plsc.subcore_barrier()
    pltpu.sync_copy(acc_sh.at[pl.ds(r0, RPT)],
                    out_hbm.at[pl.ds(cid * NPAD + r0, RPT)])


def _dinv(dp_ref):
    dp = dp_ref[...]
    return lax.rsqrt(dp[0] + dp[1] + 1.0)[:N]


def _enc_body(dp_ref, x_ref, emb_ref, w0_ref, g0_ref):
    d = _dinv(dp_ref)
    tbl = jnp.dot(emb_ref[...], w0_ref[...],
                  preferred_element_type=jnp.float32)
    onehot = (x_ref[...] ==
              lax.broadcasted_iota(jnp.int32, (N, NUM_NODE_TYPES), 1)
              ).astype(jnp.float32)
    g0_ref[...] = jnp.dot(onehot, tbl,
                          preferred_element_type=jnp.float32) * d


_enc = pl.pallas_call(
    _enc_body,
    out_shape=jax.ShapeDtypeStruct((N, HID), jnp.float32),
)


def _mid_body(s_ref, g_ref, dp_ref, b_ref, w_ref, out_ref):
    d = _dinv(dp_ref)
    s = s_ref[...]
    h = jax.nn.relu(d * (s[0, :N] + s[1, :N] + g_ref[...]) + b_ref[...])
    out_ref[...] = jnp.dot(h, w_ref[...],
                           preferred_element_type=jnp.float32) * d


_mid = pl.pallas_call(
    _mid_body,
    out_shape=jax.ShapeDtypeStruct((N, HID), jnp.float32),
)


def _head_body(s_ref, g_ref, dp_ref, b2_ref, wh0_ref, bh0_ref, wh1_ref,
               bh1_ref, y_ref, pred_ref, tc_ref):
    d = _dinv(dp_ref)
    s = s_ref[...]
    h = jax.nn.relu(d * (s[0, :N] + s[1, :N] + g_ref[...]) + b2_ref[...])
    t = jax.nn.relu(jnp.dot(h, wh0_ref[...],
                            preferred_element_type=jnp.float32) + bh0_ref[...])
    pred_ref[...] = jnp.dot(t, wh1_ref[...],
                            preferred_element_type=jnp.float32) + bh1_ref[...]
    y = y_ref[...]
    tc_ref[...] = jnp.where(y != -1, y, -1)


_head = pl.pallas_call(
    _head_body,
    out_shape=(
        jax.ShapeDtypeStruct((N, HID), jnp.float32),
        jax.ShapeDtypeStruct((N, 1), jnp.int32),
    ),
)


@jax.jit
def kernel(x, edge_index, edge_attr, y, node_emb, edge_emb,
           W0, b0, W1, b1, W2, b2, Wh0, bh0, Wh1, bh1):
    del edge_attr, edge_emb
    ei = edge_index.astype(jnp.int32)
    src = jnp.concatenate(
        [ei[0], jnp.zeros((E_PAD - E,), jnp.int32)]).reshape(NW, CHUNKS, CHUNK)
    dst = jnp.concatenate(
        [ei[1], jnp.full((E_PAD - E,), DUMMY, jnp.int32)]
    ).reshape(NW, CHUNKS, CHUNK)

    ones_c = jnp.ones((CHUNK,), jnp.float32)
    zeros_n = jnp.zeros((NPAD,), jnp.float32)
    zeros_nh = jnp.zeros((NPAD, HID), jnp.float32)

    dp = _deg_kernel(dst, ones_c, zeros_n).reshape(NC, NPAD, 1)

    x2 = x.astype(jnp.int32).reshape(N, 1)
    y2 = y.astype(jnp.int32).reshape(N, 1)
    b0r = b0.reshape(1, HID)
    b1r = b1.reshape(1, HID)
    b2r = b2.reshape(1, HID)
    bh0r = bh0.reshape(1, HID)
    wh1p = jnp.zeros((HID, HID), jnp.float32).at[:, :NUM_CLASSES].set(Wh1)
    bh1p = jnp.zeros((1, HID), jnp.float32).at[0, :NUM_CLASSES].set(bh1)

    g0 = _enc(dp, x2, node_emb, W0)
    s0 = _agg_kernel(g0, src, dst, zeros_nh).reshape(NC, NPAD, HID)
    g1 = _mid(s0, g0, dp, b0r, W1)
    s1 = _agg_kernel(g1, src, dst, zeros_nh).reshape(NC, NPAD, HID)
    g2 = _mid(s1, g1, dp, b1r, W2)
    s2 = _agg_kernel(g2, src, dst, zeros_nh).reshape(NC, NPAD, HID)
    pred_pad, tc = _head(s2, g2, dp, b2r, Wh0, bh0r, wh1p, bh1p, y2)

    pred = pred_pad[:, :NUM_CLASSES]
    true_class = tc.reshape(N)
    true_label = jnp.full((N,), -1.0, jnp.float32)
    return pred, true_class, true_label


# speedup vs baseline: 1.1721x; 1.1721x over previous
import functools

import jax
import jax.numpy as jnp
from jax import lax
from jax.experimental import pallas as pl
from jax.experimental.pallas import tpu as pltpu
from jax.experimental.pallas import tpu_sc as plsc

N = 10000
E = 320000
HID = 128
NUM_NODE_TYPES = 8
NUM_CLASSES = 10

NC = 2
NS = 16
NW = NC * NS
CHUNK = 128
CHUNKS = 79
E_PAD = NW * CHUNKS * CHUNK
NPAD = 10240
DUMMY = NPAD - 8
RPT = NPAD // NS

_mesh = plsc.VectorSubcoreMesh(
    core_axis_name="c", subcore_axis_name="s", num_cores=NC, num_subcores=NS
)


@functools.partial(
    pl.kernel,
    out_type=jax.ShapeDtypeStruct((NC * NPAD,), jnp.float32),
    mesh=_mesh,
    scratch_types=[
        pltpu.VMEM((CHUNKS, CHUNK), jnp.int32),
        pltpu.VMEM((CHUNK,), jnp.float32),
        pltpu.VMEM_SHARED((NPAD,), jnp.float32),
    ],
)
def _deg_kernel(dst_hbm, ones_hbm, zeros_hbm, out_hbm, dst_v, ones_v, deg_sh):
    cid = lax.axis_index("c")
    sid = lax.axis_index("s")
    w = cid * NS + sid
    r0 = sid * RPT
    pltpu.sync_copy(zeros_hbm.at[pl.ds(r0, RPT)], deg_sh.at[pl.ds(r0, RPT)])
    pltpu.sync_copy(ones_hbm, ones_v)
    pltpu.sync_copy(dst_hbm.at[w], dst_v)
    plsc.subcore_barrier()

    def body(j, c):
        pltpu.sync_copy(ones_v, deg_sh.at[dst_v.at[j]], add=True)
        return c

    lax.fori_loop(0, CHUNKS, body, 0)
    plsc.subcore_barrier()
    pltpu.sync_copy(deg_sh.at[pl.ds(r0, RPT)],
                    out_hbm.at[pl.ds(cid * NPAD + r0, RPT)])


@functools.partial(
    pl.kernel,
    out_type=jax.ShapeDtypeStruct((NC * NPAD, HID), jnp.float32),
    mesh=_mesh,
    scratch_types=[
        pltpu.VMEM((CHUNKS, CHUNK), jnp.int32),
        pltpu.VMEM((CHUNKS, CHUNK), jnp.int32),
        pltpu.VMEM((CHUNK, HID), jnp.float32),
        pltpu.VMEM_SHARED((NPAD, HID), jnp.float32),
        pltpu.SemaphoreType.DMA,
    ],
)
def _agg_kernel(g_hbm, src_hbm, dst_hbm, zeros_hbm, out_hbm,
                src_v, dst_v, rows_v, acc_sh, sem):
    cid = lax.axis_index("c")
    sid = lax.axis_index("s")
    w = cid * NS + sid
    r0 = sid * RPT
    pltpu.sync_copy(zeros_hbm.at[pl.ds(r0, RPT)], acc_sh.at[pl.ds(r0, RPT)])
    pltpu.sync_copy(src_hbm.at[w], src_v)
    pltpu.sync_copy(dst_hbm.at[w], dst_v)
    plsc.subcore_barrier()

    def body(j, c):
        pltpu.async_copy(g_hbm.at[src_v.at[j]], rows_v, sem).wait()
        pltpu.sync_copy(rows_v, acc_sh.at[dst_v.at[j]], add=True)
        return c

    lax.fori_loop(0, CHUNKS, body, 0)
    plsc.subcore_barrier()
    pltpu.sync_copy(acc_sh.at[pl.ds(r0, RPT)],
                    out_hbm.at[pl.ds(cid * NPAD + r0, RPT)])


NT = 8


@functools.partial(
    pl.kernel,
    out_type=jax.ShapeDtypeStruct((NC * NPAD * NT,), jnp.float32),
    mesh=_mesh,
    scratch_types=[
        pltpu.VMEM((CHUNKS, CHUNK), jnp.int32),
        pltpu.VMEM((CHUNKS, CHUNK), jnp.int32),
        pltpu.VMEM((CHUNK,), jnp.float32),
        pltpu.VMEM((CHUNK,), jnp.int32),
        pltpu.VMEM((CHUNK,), jnp.int32),
        pltpu.VMEM_SHARED((NPAD * NT,), jnp.float32),
        pltpu.SemaphoreType.DMA,
    ],
)
def _t8_kernel(d_hbm, x_hbm, src_hbm, dst_hbm, zeros_hbm, out_hbm,
               src_v, dst_v, dg, xg, sidx, t_sh, sem):
    cid = lax.axis_index("c")
    sid = lax.axis_index("s")
    w = cid * NS + sid
    r0 = sid * RPT * NT
    pltpu.sync_copy(zeros_hbm.at[pl.ds(r0, RPT * NT)],
                    t_sh.at[pl.ds(r0, RPT * NT)])
    pltpu.sync_copy(src_hbm.at[w], src_v)
    pltpu.sync_copy(dst_hbm.at[w], dst_v)
    plsc.subcore_barrier()

    def body(j, c):
        pltpu.async_copy(d_hbm.at[src_v.at[j]], dg, sem).wait()
        pltpu.async_copy(x_hbm.at[src_v.at[j]], xg, sem).wait()
        dj = dst_v.at[j]
        for k in range(CHUNK // 16):
            sl = pl.ds(k * 16, 16)
            sidx[sl] = dj[sl] * NT + xg[sl]
        pltpu.sync_copy(dg, t_sh.at[sidx], add=True)
        return c

    lax.fori_loop(0, CHUNKS, body, 0)
    plsc.subcore_barrier()
    pltpu.sync_copy(t_sh.at[pl.ds(r0, RPT * NT)],
                    out_hbm.at[pl.ds(cid * NPAD * NT + r0, RPT * NT)])


def _dinv(dp_ref):
    dp = dp_ref[...]
    return lax.rsqrt(dp[0] + dp[1] + 1.0)[:N]


def _onehot(x_ref):
    return (x_ref[...] ==
            lax.broadcasted_iota(jnp.int32, (N, NUM_NODE_TYPES), 1)
            ).astype(jnp.float32)


def _dv_body(dp_ref, dv_ref):
    d = _dinv(dp_ref)
    dv_ref[pl.ds(0, N)] = d
    dv_ref[pl.ds(N, NPAD - N)] = jnp.zeros((NPAD - N, 1), jnp.float32)


_dv = pl.pallas_call(
    _dv_body,
    out_shape=jax.ShapeDtypeStruct((NPAD, 1), jnp.float32),
)


def _lay1_body(t_ref, dp_ref, x_ref, emb_ref, w0_ref, b_ref, w1_ref, out_ref):
    d = _dinv(dp_ref)
    tbl = jnp.dot(emb_ref[...], w0_ref[...],
                  preferred_element_type=jnp.float32)
    t = t_ref[...]
    tsum = t[0, :N] + t[1, :N]
    s1 = jnp.dot(tsum, tbl, preferred_element_type=jnp.float32)
    g0 = jnp.dot(_onehot(x_ref), tbl,
                 preferred_element_type=jnp.float32) * d
    h1 = jax.nn.relu(d * (s1 + g0) + b_ref[...])
    out_ref[...] = jnp.dot(h1, w1_ref[...],
                           preferred_element_type=jnp.float32) * d


_lay1 = pl.pallas_call(
    _lay1_body,
    out_shape=jax.ShapeDtypeStruct((N, HID), jnp.float32),
)


def _mid_body(s_ref, g_ref, dp_ref, b_ref, w_ref, out_ref):
    d = _dinv(dp_ref)
    s = s_ref[...]
    h = jax.nn.relu(d * (s[0, :N] + s[1, :N] + g_ref[...]) + b_ref[...])
    out_ref[...] = jnp.dot(h, w_ref[...],
                           preferred_element_type=jnp.float32) * d


_mid = pl.pallas_call(
    _mid_body,
    out_shape=jax.ShapeDtypeStruct((N, HID), jnp.float32),
)


def _head_body(s_ref, g_ref, dp_ref, b2_ref, wh0_ref, bh0_ref, wh1_ref,
               bh1_ref, y_ref, pred_ref, tc_ref):
    d = _dinv(dp_ref)
    s = s_ref[...]
    h = jax.nn.relu(d * (s[0, :N] + s[1, :N] + g_ref[...]) + b2_ref[...])
    t = jax.nn.relu(jnp.dot(h, wh0_ref[...],
                            preferred_element_type=jnp.float32) + bh0_ref[...])
    pred_ref[...] = jnp.dot(t, wh1_ref[...],
                            preferred_element_type=jnp.float32) + bh1_ref[...]
    y = y_ref[...]
    tc_ref[...] = jnp.where(y != -1, y, -1)


_head = pl.pallas_call(
    _head_body,
    out_shape=(
        jax.ShapeDtypeStruct((N, HID), jnp.float32),
        jax.ShapeDtypeStruct((N, 1), jnp.int32),
    ),
)


@jax.jit
def kernel(x, edge_index, edge_attr, y, node_emb, edge_emb,
           W0, b0, W1, b1, W2, b2, Wh0, bh0, Wh1, bh1):
    del edge_attr, edge_emb
    ei = edge_index.astype(jnp.int32)
    src = jnp.concatenate(
        [ei[0], jnp.zeros((E_PAD - E,), jnp.int32)]).reshape(NW, CHUNKS, CHUNK)
    dst = jnp.concatenate(
        [ei[1], jnp.full((E_PAD - E,), DUMMY, jnp.int32)]
    ).reshape(NW, CHUNKS, CHUNK)

    ones_c = jnp.ones((CHUNK,), jnp.float32)
    zeros_n = jnp.zeros((NPAD,), jnp.float32)
    zeros_nh = jnp.zeros((NPAD, HID), jnp.float32)
    zeros_t8 = jnp.zeros((NPAD * NT,), jnp.float32)

    dp = _deg_kernel(dst, ones_c, zeros_n).reshape(NC, NPAD, 1)

    x2 = x.astype(jnp.int32).reshape(N, 1)
    y2 = y.astype(jnp.int32).reshape(N, 1)
    b0r = b0.reshape(1, HID)
    b1r = b1.reshape(1, HID)
    b2r = b2.reshape(1, HID)
    bh0r = bh0.reshape(1, HID)
    wh1p = jnp.zeros((HID, HID), jnp.float32).at[:, :NUM_CLASSES].set(Wh1)
    bh1p = jnp.zeros((1, HID), jnp.float32).at[0, :NUM_CLASSES].set(bh1)

    dv = _dv(dp).reshape(NPAD)
    xpad = jnp.concatenate([x2.reshape(N), jnp.zeros((NPAD - N,), jnp.int32)])
    t8 = _t8_kernel(dv, xpad, src, dst, zeros_t8).reshape(NC, NPAD, NT)
    g1 = _lay1(t8, dp, x2, node_emb, W0, b0r, W1)
    s1 = _agg_kernel(g1, src, dst, zeros_nh).reshape(NC, NPAD, HID)
    g2 = _mid(s1, g1, dp, b1r, W2)
    s2 = _agg_kernel(g2, src, dst, zeros_nh).reshape(NC, NPAD, HID)
    pred_pad, tc = _head(s2, g2, dp, b2r, Wh0, bh0r, wh1p, bh1p, y2)

    pred = pred_pad[:, :NUM_CLASSES]
    true_class = tc.reshape(N)
    true_label = jnp.full((N,), -1.0, jnp.float32)
    return pred, true_class, true_label


# speedup vs baseline: 1.2156x; 1.0371x over previous
import functools

import jax
import jax.numpy as jnp
from jax import lax
from jax.experimental import pallas as pl
from jax.experimental.pallas import tpu as pltpu
from jax.experimental.pallas import tpu_sc as plsc

N = 10000
E = 320000
HID = 128
NUM_NODE_TYPES = 8
NUM_CLASSES = 10

NC = 2
NS = 16
NW = NC * NS
CHUNK = 128
CHUNKS = 79
E_PAD = NW * CHUNKS * CHUNK
NPAD = 10240
DUMMY = NPAD - 8
RPT = NPAD // NS

_mesh = plsc.VectorSubcoreMesh(
    core_axis_name="c", subcore_axis_name="s", num_cores=NC, num_subcores=NS
)


@functools.partial(
    pl.kernel,
    out_type=jax.ShapeDtypeStruct((NC * NPAD,), jnp.float32),
    mesh=_mesh,
    scratch_types=[
        pltpu.VMEM((CHUNKS, CHUNK), jnp.int32),
        pltpu.VMEM((CHUNK,), jnp.float32),
        pltpu.VMEM_SHARED((NPAD,), jnp.float32),
    ],
)
def _deg_kernel(dst_hbm, ones_hbm, zeros_hbm, out_hbm, dst_v, ones_v, deg_sh):
    cid = lax.axis_index("c")
    sid = lax.axis_index("s")
    w = cid * NS + sid
    r0 = sid * RPT
    pltpu.sync_copy(zeros_hbm.at[pl.ds(r0, RPT)], deg_sh.at[pl.ds(r0, RPT)])
    pltpu.sync_copy(ones_hbm, ones_v)
    pltpu.sync_copy(dst_hbm.at[w], dst_v)
    plsc.subcore_barrier()

    def body(j, c):
        pltpu.sync_copy(ones_v, deg_sh.at[dst_v.at[j]], add=True)
        return c

    lax.fori_loop(0, CHUNKS, body, 0)
    plsc.subcore_barrier()
    pltpu.sync_copy(deg_sh.at[pl.ds(r0, RPT)],
                    out_hbm.at[pl.ds(cid * NPAD + r0, RPT)])


@functools.partial(
    pl.kernel,
    out_type=jax.ShapeDtypeStruct((NC * NPAD, HID), jnp.float32),
    mesh=_mesh,
    scratch_types=[
        pltpu.VMEM((CHUNKS, CHUNK), jnp.int32),
        pltpu.VMEM((CHUNKS, CHUNK), jnp.int32),
        pltpu.VMEM((CHUNK, HID), jnp.float32),
        pltpu.VMEM_SHARED((NPAD, HID), jnp.float32),
        pltpu.SemaphoreType.DMA,
    ],
)
def _agg_kernel(g_hbm, src_hbm, dst_hbm, zeros_hbm, out_hbm,
                src_v, dst_v, rows_v, acc_sh, sem):
    cid = lax.axis_index("c")
    sid = lax.axis_index("s")
    w = cid * NS + sid
    r0 = sid * RPT
    pltpu.sync_copy(zeros_hbm.at[pl.ds(r0, RPT)], acc_sh.at[pl.ds(r0, RPT)])
    pltpu.sync_copy(src_hbm.at[w], src_v)
    pltpu.sync_copy(dst_hbm.at[w], dst_v)
    plsc.subcore_barrier()

    def body(j, c):
        pltpu.async_copy(g_hbm.at[src_v.at[j]], rows_v, sem).wait()
        pltpu.sync_copy(rows_v, acc_sh.at[dst_v.at[j]], add=True)
        return c

    lax.fori_loop(0, CHUNKS, body, 0)
    plsc.subcore_barrier()
    pltpu.sync_copy(acc_sh.at[pl.ds(r0, RPT)],
                    out_hbm.at[pl.ds(cid * NPAD + r0, RPT)])


NT = 8


@functools.partial(
    pl.kernel,
    out_type=jax.ShapeDtypeStruct((NC * NPAD * NT,), jnp.float32),
    mesh=_mesh,
    scratch_types=[
        pltpu.VMEM((CHUNKS, CHUNK), jnp.int32),
        pltpu.VMEM((CHUNKS, CHUNK), jnp.int32),
        pltpu.VMEM((CHUNK,), jnp.float32),
        pltpu.VMEM((CHUNK,), jnp.float32),
        pltpu.VMEM((CHUNK,), jnp.int32),
        pltpu.VMEM_SHARED((NPAD * NT,), jnp.float32),
        pltpu.SemaphoreType.DMA,
    ],
)
def _t8_kernel(p_hbm, src_hbm, dst_hbm, zeros_hbm, out_hbm,
               src_v, dst_v, pg, dg, sidx, t_sh, sem):
    cid = lax.axis_index("c")
    sid = lax.axis_index("s")
    w = cid * NS + sid
    r0 = sid * RPT * NT
    pltpu.sync_copy(zeros_hbm.at[pl.ds(r0, RPT * NT)],
                    t_sh.at[pl.ds(r0, RPT * NT)])
    pltpu.sync_copy(src_hbm.at[w], src_v)
    pltpu.sync_copy(dst_hbm.at[w], dst_v)
    plsc.subcore_barrier()

    def body(j, c):
        pltpu.async_copy(p_hbm.at[src_v.at[j]], pg, sem).wait()
        dj = dst_v.at[j]
        for k in range(CHUNK // 16):
            sl = pl.ds(k * 16, 16)
            pk = pg[sl]
            xi = (pk * 0.25).astype(jnp.int32)
            dg[sl] = pk - (xi * 4).astype(jnp.float32)
            sidx[sl] = dj[sl] * NT + xi
        pltpu.sync_copy(dg, t_sh.at[sidx], add=True)
        return c

    lax.fori_loop(0, CHUNKS, body, 0)
    plsc.subcore_barrier()
    pltpu.sync_copy(t_sh.at[pl.ds(r0, RPT * NT)],
                    out_hbm.at[pl.ds(cid * NPAD * NT + r0, RPT * NT)])


def _dinv(dp_ref):
    dp = dp_ref[...]
    return lax.rsqrt(dp[0] + dp[1] + 1.0)[:N]


def _onehot(x_ref):
    return (x_ref[...] ==
            lax.broadcasted_iota(jnp.int32, (N, NUM_NODE_TYPES), 1)
            ).astype(jnp.float32)


def _dv_body(dp_ref, x_ref, dv_ref):
    d = _dinv(dp_ref)
    p = x_ref[...].astype(jnp.float32) * 4.0 + d
    dv_ref[pl.ds(0, N)] = p
    dv_ref[pl.ds(N, NPAD - N)] = jnp.zeros((NPAD - N, 1), jnp.float32)


_dv = pl.pallas_call(
    _dv_body,
    out_shape=jax.ShapeDtypeStruct((NPAD, 1), jnp.float32),
)


def _lay1_body(t_ref, dp_ref, x_ref, emb_ref, w0_ref, b_ref, w1_ref, out_ref):
    d = _dinv(dp_ref)
    tbl = jnp.dot(emb_ref[...], w0_ref[...],
                  preferred_element_type=jnp.float32)
    t = t_ref[...]
    tsum = t[0, :N] + t[1, :N]
    s1 = jnp.dot(tsum, tbl, preferred_element_type=jnp.float32)
    g0 = jnp.dot(_onehot(x_ref), tbl,
                 preferred_element_type=jnp.float32) * d
    h1 = jax.nn.relu(d * (s1 + g0) + b_ref[...])
    out_ref[...] = jnp.dot(h1, w1_ref[...],
                           preferred_element_type=jnp.float32) * d


_lay1 = pl.pallas_call(
    _lay1_body,
    out_shape=jax.ShapeDtypeStruct((N, HID), jnp.float32),
)


def _mid_body(s_ref, g_ref, dp_ref, b_ref, w_ref, out_ref):
    d = _dinv(dp_ref)
    s = s_ref[...]
    h = jax.nn.relu(d * (s[0, :N] + s[1, :N] + g_ref[...]) + b_ref[...])
    out_ref[...] = jnp.dot(h, w_ref[...],
                           preferred_element_type=jnp.float32) * d


_mid = pl.pallas_call(
    _mid_body,
    out_shape=jax.ShapeDtypeStruct((N, HID), jnp.float32),
)


def _head_body(s_ref, g_ref, dp_ref, b2_ref, wh0_ref, bh0_ref, wh1_ref,
               bh1_ref, y_ref, pred_ref, tc_ref):
    d = _dinv(dp_ref)
    s = s_ref[...]
    h = jax.nn.relu(d * (s[0, :N] + s[1, :N] + g_ref[...]) + b2_ref[...])
    t = jax.nn.relu(jnp.dot(h, wh0_ref[...],
                            preferred_element_type=jnp.float32) + bh0_ref[...])
    pred_ref[...] = jnp.dot(t, wh1_ref[...],
                            preferred_element_type=jnp.float32) + bh1_ref[...]
    y = y_ref[...]
    tc_ref[...] = jnp.where(y != -1, y, -1)


_head = pl.pallas_call(
    _head_body,
    out_shape=(
        jax.ShapeDtypeStruct((N, HID), jnp.float32),
        jax.ShapeDtypeStruct((N, 1), jnp.int32),
    ),
)


@jax.jit
def kernel(x, edge_index, edge_attr, y, node_emb, edge_emb,
           W0, b0, W1, b1, W2, b2, Wh0, bh0, Wh1, bh1):
    del edge_attr, edge_emb
    ei = edge_index.astype(jnp.int32)
    src = jnp.concatenate(
        [ei[0], jnp.zeros((E_PAD - E,), jnp.int32)]).reshape(NW, CHUNKS, CHUNK)
    dst = jnp.concatenate(
        [ei[1], jnp.full((E_PAD - E,), DUMMY, jnp.int32)]
    ).reshape(NW, CHUNKS, CHUNK)

    ones_c = jnp.ones((CHUNK,), jnp.float32)
    zeros_n = jnp.zeros((NPAD,), jnp.float32)
    zeros_nh = jnp.zeros((NPAD, HID), jnp.float32)
    zeros_t8 = jnp.zeros((NPAD * NT,), jnp.float32)

    dp = _deg_kernel(dst, ones_c, zeros_n).reshape(NC, NPAD, 1)

    x2 = x.astype(jnp.int32).reshape(N, 1)
    y2 = y.astype(jnp.int32).reshape(N, 1)
    b0r = b0.reshape(1, HID)
    b1r = b1.reshape(1, HID)
    b2r = b2.reshape(1, HID)
    bh0r = bh0.reshape(1, HID)
    wh1p = jnp.zeros((HID, HID), jnp.float32).at[:, :NUM_CLASSES].set(Wh1)
    bh1p = jnp.zeros((1, HID), jnp.float32).at[0, :NUM_CLASSES].set(bh1)

    pv = _dv(dp, x2).reshape(NPAD)
    t8 = _t8_kernel(pv, src, dst, zeros_t8).reshape(NC, NPAD, NT)
    g1 = _lay1(t8, dp, x2, node_emb, W0, b0r, W1)
    s1 = _agg_kernel(g1, src, dst, zeros_nh).reshape(NC, NPAD, HID)
    g2 = _mid(s1, g1, dp, b1r, W2)
    s2 = _agg_kernel(g2, src, dst, zeros_nh).reshape(NC, NPAD, HID)
    pred_pad, tc = _head(s2, g2, dp, b2r, Wh0, bh0r, wh1p, bh1p, y2)

    pred = pred_pad[:, :NUM_CLASSES]
    true_class = tc.reshape(N)
    true_label = jnp.full((N,), -1.0, jnp.float32)
    return pred, true_class, true_label
